# baseline (speedup 1.0000x reference)
import functools

import numpy as np
import jax
import jax.numpy as jnp
from jax import lax
from jax.experimental import pallas as pl
from jax.experimental.pallas import tpu as pltpu
from jax.experimental.pallas import tpu_sc as plsc

_EMB = 64
_SEQ = 50
_BATCH = 16384
_NPARAM = 10000
_B = _BATCH * _SEQ
_NW = 32
_PER_W = _B // _NW
_W = 800
_NCHUNK = _PER_W // _W
_SUB = 80
_NSUB = _W // _SUB
_SENT_BLK = 16
_TC_GRID = _BATCH // _SENT_BLK


def _pos_table():
    pos = np.arange(_SEQ, dtype=np.float64)[:, None]
    dim = np.arange(_EMB // 2, dtype=np.float64)[None, :]
    theta = pos / (_NPARAM ** (2.0 * dim / _EMB))
    pe = np.zeros((_SEQ, _EMB), dtype=np.float64)
    pe[:, 0::2] = np.sin(theta)
    pe[:, 1::2] = np.cos(theta)
    return pe.astype(np.float32)


_POS = _pos_table()


def _sc_gather(idx_flat, table):
    mesh = plsc.VectorSubcoreMesh(
        core_axis_name="c", subcore_axis_name="s", num_cores=2, num_subcores=16
    )

    @functools.partial(
        pl.kernel,
        out_type=jax.ShapeDtypeStruct((_B, _EMB), jnp.float32),
        mesh=mesh,
        scratch_types=[
            pltpu.VMEM((_W,), jnp.int32),
            pltpu.VMEM((_W, _EMB), jnp.float32),
            pltpu.SemaphoreType.DMA,
        ],
        compiler_params=pltpu.CompilerParams(use_tc_tiling_on_sc=False),
    )
    def k(idx_hbm, tab_hbm, g_hbm, idx_v, rows_v, gsem):
        wid = lax.axis_index("s") * 2 + lax.axis_index("c")
        base0 = wid * _PER_W

        @pl.loop(0, _NCHUNK)
        def _chunk_loop(chunk):
            base = base0 + chunk * _W
            pltpu.sync_copy(idx_hbm.at[pl.ds(base, _W)], idx_v)
            copies = [
                pltpu.async_copy(
                    tab_hbm.at[idx_v.at[pl.ds(j * _SUB, _SUB)]],
                    rows_v.at[pl.ds(j * _SUB, _SUB)],
                    gsem,
                )
                for j in range(_NSUB)
            ]
            for cp in copies:
                cp.wait()
            pltpu.sync_copy(rows_v, g_hbm.at[pl.ds(base, _W)])

    return k(idx_flat, table)


def _tc_finish(g, pos_even, pos_odd):
    def body(g_ref, pe_ref, po_ref, o_ref):
        g3 = g_ref[...].reshape(_SENT_BLK, _SEQ // 2, 128)
        o_ref[:, 0::2, :] = g3[:, :, :_EMB] * 8.0 + pe_ref[...][None, :, :]
        o_ref[:, 1::2, :] = g3[:, :, _EMB:] * 8.0 + po_ref[...][None, :, :]

    rows_blk = _SENT_BLK * _SEQ // 2
    return pl.pallas_call(
        body,
        out_shape=jax.ShapeDtypeStruct((_BATCH, _SEQ, _EMB), jnp.float32),
        grid=(_TC_GRID,),
        in_specs=[
            pl.BlockSpec((rows_blk, 128), lambda i: (i, 0)),
            pl.BlockSpec((_SEQ // 2, _EMB), lambda i: (0, 0)),
            pl.BlockSpec((_SEQ // 2, _EMB), lambda i: (0, 0)),
        ],
        out_specs=pl.BlockSpec((_SENT_BLK, _SEQ, _EMB), lambda i: (i, 0, 0)),
    )(g, pos_even, pos_odd)


def _tc_mask(tok):
    def body(t_ref, m_ref):
        m_ref[...] = t_ref[...] != 0

    return pl.pallas_call(
        body,
        out_shape=jax.ShapeDtypeStruct((_BATCH, _SEQ), jnp.bool_),
        grid=(16,),
        in_specs=[pl.BlockSpec((_BATCH // 16, _SEQ), lambda i: (i, 0))],
        out_specs=pl.BlockSpec((_BATCH // 16, _SEQ), lambda i: (i, 0)),
    )(tok)


def kernel(tokenized_sentences, embedding_table):
    tok = tokenized_sentences.astype(jnp.int32)
    idx_flat = tok.reshape(_B)
    g = _sc_gather(idx_flat, embedding_table).reshape(_B // 2, 128)
    enc = _tc_finish(g, jnp.asarray(_POS[0::2]), jnp.asarray(_POS[1::2]))
    mask = _tc_mask(tok)
    return enc, mask


# speedup vs baseline: 1.2872x; 1.2872x over previous
import functools

import numpy as np
import jax
import jax.numpy as jnp
from jax import lax
from jax.experimental import pallas as pl
from jax.experimental.pallas import tpu as pltpu
from jax.experimental.pallas import tpu_sc as plsc

_EMB = 64
_SEQ = 50
_BATCH = 16384
_NPARAM = 10000
_B = _BATCH * _SEQ
_NW = 32
_PER_W = _B // _NW
_W = 800
_NCHUNK = _PER_W // _W
_SUB = 80
_NSUB = _W // _SUB
_SENT_BLK = 64
_TC_GRID = _BATCH // _SENT_BLK

_PERM = np.array([(j % 2) * 25 + j // 2 for j in range(_SEQ)], dtype=np.int32)


def _pos_table():
    pos = np.arange(_SEQ, dtype=np.float64)[:, None]
    dim = np.arange(_EMB // 2, dtype=np.float64)[None, :]
    theta = pos / (_NPARAM ** (2.0 * dim / _EMB))
    pe = np.zeros((_SEQ, _EMB), dtype=np.float64)
    pe[:, 0::2] = np.sin(theta)
    pe[:, 1::2] = np.cos(theta)
    return pe.astype(np.float32)


_POS = _pos_table()


def _sc_gather(idx_flat, table):
    mesh = plsc.VectorSubcoreMesh(
        core_axis_name="c", subcore_axis_name="s", num_cores=2, num_subcores=16
    )

    @functools.partial(
        pl.kernel,
        out_type=jax.ShapeDtypeStruct((_B, _EMB), jnp.float32),
        mesh=mesh,
        scratch_types=[
            pltpu.VMEM((_W,), jnp.int32),
            pltpu.VMEM((_W, _EMB), jnp.float32),
            pltpu.SemaphoreType.DMA,
        ],
        compiler_params=pltpu.CompilerParams(use_tc_tiling_on_sc=False),
    )
    def k(idx_hbm, tab_hbm, g_hbm, idx_v, rows_v, gsem):
        wid = lax.axis_index("s") * 2 + lax.axis_index("c")
        base0 = wid * _PER_W

        @pl.loop(0, _NCHUNK)
        def _chunk_loop(chunk):
            base = base0 + chunk * _W
            pltpu.sync_copy(idx_hbm.at[pl.ds(base, _W)], idx_v)
            copies = [
                pltpu.async_copy(
                    tab_hbm.at[idx_v.at[pl.ds(j * _SUB, _SUB)]],
                    rows_v.at[pl.ds(j * _SUB, _SUB)],
                    gsem,
                )
                for j in range(_NSUB)
            ]
            for cp in copies:
                cp.wait()
            pltpu.sync_copy(rows_v, g_hbm.at[pl.ds(base, _W)])

    return k(idx_flat, table)


def _tc_finish(g, pos_pair):
    def body(g_ref, p_ref, o_ref):
        z = g_ref[...].reshape(_SENT_BLK, _SEQ // 2, 128) * 8.0 + p_ref[...][None, :, :]
        o_ref[:, : _SEQ // 2, :] = z[:, :, :_EMB]
        o_ref[:, _SEQ // 2 :, :] = z[:, :, _EMB:]

    rows_blk = _SENT_BLK * _SEQ // 2
    return pl.pallas_call(
        body,
        out_shape=jax.ShapeDtypeStruct((_BATCH, _SEQ, _EMB), jnp.float32),
        grid=(_TC_GRID,),
        in_specs=[
            pl.BlockSpec((rows_blk, 128), lambda i: (i, 0)),
            pl.BlockSpec((_SEQ // 2, 128), lambda i: (0, 0)),
        ],
        out_specs=pl.BlockSpec((_SENT_BLK, _SEQ, _EMB), lambda i: (i, 0, 0)),
    )(g, pos_pair)


def _tc_mask(tok):
    def body(t_ref, m_ref):
        m_ref[...] = t_ref[...] != 0

    return pl.pallas_call(
        body,
        out_shape=jax.ShapeDtypeStruct((_BATCH, _SEQ), jnp.bool_),
        grid=(16,),
        in_specs=[pl.BlockSpec((_BATCH // 16, _SEQ), lambda i: (i, 0))],
        out_specs=pl.BlockSpec((_BATCH // 16, _SEQ), lambda i: (i, 0)),
    )(tok)


def kernel(tokenized_sentences, embedding_table):
    tok = tokenized_sentences.astype(jnp.int32)
    idx_flat = tok[:, _PERM].reshape(_B)
    g = _sc_gather(idx_flat, embedding_table).reshape(_B // 2, 128)
    pos_pair = np.concatenate([_POS[: _SEQ // 2], _POS[_SEQ // 2 :]], axis=1)
    enc = _tc_finish(g, jnp.asarray(pos_pair))
    mask = _tc_mask(tok)
    return enc, mask


# speedup vs baseline: 1.7651x; 1.3713x over previous
import functools

import numpy as np
import jax
import jax.numpy as jnp
from jax import lax
from jax.experimental import pallas as pl
from jax.experimental.pallas import tpu as pltpu
from jax.experimental.pallas import tpu_sc as plsc

_EMB = 64
_SEQ = 50
_BATCH = 16384
_NPARAM = 10000
_B = _BATCH * _SEQ
_NW = 32
_PER_W = _B // _NW
_W = 800
_NCHUNK = _PER_W // _W
_SUB = 80
_NSUB = _W // _SUB
_SENT_BLK = 256
_TC_GRID = _BATCH // _SENT_BLK

_PERM = np.array([(j % 2) * 25 + j // 2 for j in range(_SEQ)], dtype=np.int32)


def _pos_table():
    pos = np.arange(_SEQ, dtype=np.float64)[:, None]
    dim = np.arange(_EMB // 2, dtype=np.float64)[None, :]
    theta = pos / (_NPARAM ** (2.0 * dim / _EMB))
    pe = np.zeros((_SEQ, _EMB), dtype=np.float64)
    pe[:, 0::2] = np.sin(theta)
    pe[:, 1::2] = np.cos(theta)
    return pe.astype(np.float32)


_POS = _pos_table()


def _sc_gather(idx_flat, table):
    mesh = plsc.VectorSubcoreMesh(
        core_axis_name="c", subcore_axis_name="s", num_cores=2, num_subcores=16
    )

    @functools.partial(
        pl.kernel,
        out_type=jax.ShapeDtypeStruct((_B, _EMB), jnp.float32),
        mesh=mesh,
        scratch_types=[
            pltpu.VMEM((_W,), jnp.int32),
            pltpu.VMEM((_W, _EMB), jnp.float32),
            pltpu.SemaphoreType.DMA,
        ],
        compiler_params=pltpu.CompilerParams(use_tc_tiling_on_sc=False),
    )
    def k(idx_hbm, tab_hbm, g_hbm, idx_v, rows_v, gsem):
        wid = lax.axis_index("s") * 2 + lax.axis_index("c")
        base0 = wid * _PER_W

        @pl.loop(0, _NCHUNK)
        def _chunk_loop(chunk):
            base = base0 + chunk * _W
            pltpu.sync_copy(idx_hbm.at[pl.ds(base, _W)], idx_v)
            copies = [
                pltpu.async_copy(
                    tab_hbm.at[idx_v.at[pl.ds(j * _SUB, _SUB)]],
                    rows_v.at[pl.ds(j * _SUB, _SUB)],
                    gsem,
                )
                for j in range(_NSUB)
            ]
            for cp in copies:
                cp.wait()
            pltpu.sync_copy(rows_v, g_hbm.at[pl.ds(base, _W)])

    return k(idx_flat, table)


def _tc_finish(g, pos_pair):
    def body(g_ref, p_ref, o_ref):
        z = g_ref[...].reshape(_SENT_BLK, _SEQ // 2, 128) * 8.0 + p_ref[...][None, :, :]
        for q in range(_SEQ // 2):
            zq_t = z[:, q, :].T
            o_ref[q] = zq_t[:_EMB, :]
            o_ref[q + _SEQ // 2] = zq_t[_EMB:, :]

    rows_blk = _SENT_BLK * _SEQ // 2
    return pl.pallas_call(
        body,
        out_shape=jax.ShapeDtypeStruct((_SEQ, _EMB, _BATCH), jnp.float32),
        grid=(_TC_GRID,),
        in_specs=[
            pl.BlockSpec((rows_blk, 128), lambda i: (i, 0)),
            pl.BlockSpec((_SEQ // 2, 128), lambda i: (0, 0)),
        ],
        out_specs=pl.BlockSpec((_SEQ, _EMB, _SENT_BLK), lambda i: (0, 0, i)),
    )(g, pos_pair)


def _tc_mask(tok):
    def body(t_ref, m_ref):
        m_ref[...] = t_ref[...] != 0

    return pl.pallas_call(
        body,
        out_shape=jax.ShapeDtypeStruct((_BATCH, _SEQ), jnp.bool_),
        grid=(16,),
        in_specs=[pl.BlockSpec((_BATCH // 16, _SEQ), lambda i: (i, 0))],
        out_specs=pl.BlockSpec((_BATCH // 16, _SEQ), lambda i: (i, 0)),
    )(tok)


def kernel(tokenized_sentences, embedding_table):
    tok = tokenized_sentences.astype(jnp.int32)
    idx_flat = tok[:, _PERM].reshape(_B)
    g = _sc_gather(idx_flat, embedding_table).reshape(_B // 2, 128)
    pos_pair = np.concatenate([_POS[: _SEQ // 2], _POS[_SEQ // 2 :]], axis=1)
    enc_t = _tc_finish(g, jnp.asarray(pos_pair))
    enc = jnp.transpose(enc_t, (2, 0, 1))
    mask = _tc_mask(tok)
    return enc, mask
